# baseline (speedup 1.0000x reference)
import functools

import jax
import jax.numpy as jnp
from jax import lax
from jax.experimental import pallas as pl
from jax.experimental.pallas import tpu as pltpu
from jax.experimental.pallas import tpu_sc as plsc

NC = 2
NS = 16
CHUNK = 80


def _pad_nodes(n):
    m = NS * 16
    return (n + m - 1) // m * m


def _sc_mesh():
    return plsc.VectorSubcoreMesh(
        core_axis_name="c", subcore_axis_name="s", num_cores=NC, num_subcores=NS
    )


def _sc_degrees(edge_flat, n_nodes):
    E = edge_flat.shape[0] // 2
    ept = E // NS
    CH = 128
    full = ept // CH
    tail = ept - full * CH
    NP = _pad_nodes(n_nodes)
    rpt = NP // NS

    @functools.partial(
        pl.kernel,
        out_type=jax.ShapeDtypeStruct((NC * NP,), jnp.float32),
        mesh=_sc_mesh(),
        scratch_types=[
            pltpu.VMEM((CH,), jnp.int32),
            pltpu.VMEM((CH,), jnp.int32),
            pltpu.VMEM((CH,), jnp.int32),
            pltpu.VMEM((CH,), jnp.int32),
            pltpu.VMEM((tail,), jnp.int32) if tail else None,
            pltpu.VMEM((CH,), jnp.float32),
            pltpu.VMEM((rpt,), jnp.float32),
            pltpu.VMEM_SHARED((NP,), jnp.float32),
            pltpu.SemaphoreType.DMA,
            pltpu.SemaphoreType.DMA,
            pltpu.SemaphoreType.DMA,
            pltpu.SemaphoreType.DMA,
            pltpu.SemaphoreType.DMA,
            pltpu.SemaphoreType.DMA,
            pltpu.SemaphoreType.DMA,
            pltpu.SemaphoreType.DMA,
        ],
    )
    def deg_kernel(
        edges, out,
        i0, i1, i2, i3, it, ones_v, buf_v, deg_sh,
        sl0, sl1, sl2, sl3, ss0, ss1, ss2, ss3,
    ):
        cid = lax.axis_index("c")
        sid = lax.axis_index("s")
        ebase = cid * E + sid * ept
        ibufs = (i0, i1, i2, i3)
        slods = (sl0, sl1, sl2, sl3)
        sscat = (ss0, ss1, ss2, ss3)

        def fill(j, carry):
            ones_v[pl.ds(j * 16, 16)] = jnp.ones((16,), jnp.float32)
            return carry

        lax.fori_loop(0, CH // 16, fill, 0)

        def zbody(j, carry):
            buf_v[pl.ds(j * 16, 16)] = jnp.zeros((16,), jnp.float32)
            return carry

        lax.fori_loop(0, rpt // 16, zbody, 0)
        pltpu.sync_copy(buf_v, deg_sh.at[pl.ds(sid * rpt, rpt)])
        plsc.subcore_barrier()

        def start_l(i, b):
            pltpu.async_copy(
                edges.at[pl.ds(ebase + i * CH, CH)], ibufs[b], slods[b]
            )

        def wait_l(b):
            pltpu.make_async_copy(edges.at[pl.ds(0, CH)], ibufs[b], slods[b]).wait()

        for j in range(4):
            start_l(j, j)

        def slot(i, b):
            wait_l(b)
            pltpu.sync_copy(ones_v, deg_sh.at[ibufs[b]], add=True)

            @pl.when(i + 4 < full)
            def _():
                start_l(i + 4, b)

        def quad(q, carry):
            i = 4 * q
            slot(i, 0)
            slot(i + 1, 1)
            slot(i + 2, 2)
            slot(i + 3, 3)
            return carry

        lax.fori_loop(0, full // 4, quad, 0)
        if tail:
            pltpu.sync_copy(edges.at[pl.ds(ebase + full * CH, tail)], it)
            pltpu.sync_copy(ones_v.at[pl.ds(0, tail)], deg_sh.at[it], add=True)

        plsc.subcore_barrier()
        pltpu.sync_copy(deg_sh.at[pl.ds(sid * rpt, rpt)], buf_v)
        pltpu.sync_copy(buf_v, out.at[pl.ds(cid * NP + sid * rpt, rpt)])

    assert full % 4 == 0
    return deg_kernel(edge_flat)


def _sc_aggregate(feats, edge_flat):
    n_nodes, D = feats.shape
    E = edge_flat.shape[0] // 2
    epw = E // (NC * NS)
    CH = 72
    full = epw // CH
    tail = epw - full * CH
    NP = _pad_nodes(n_nodes)
    rpt = NP // NS

    @functools.partial(
        pl.kernel,
        out_type=jax.ShapeDtypeStruct((NC, NP, D), jnp.float32),
        mesh=_sc_mesh(),
        scratch_types=[
            pltpu.VMEM((epw,), jnp.int32),
            pltpu.VMEM((CH,), jnp.int32),
            pltpu.VMEM((CH,), jnp.int32),
            pltpu.VMEM((CH,), jnp.int32),
            pltpu.VMEM((CH,), jnp.int32),
            pltpu.VMEM((tail,), jnp.int32),
            pltpu.VMEM((CH, D), jnp.float32),
            pltpu.VMEM((CH, D), jnp.float32),
            pltpu.VMEM((CH, D), jnp.float32),
            pltpu.VMEM((CH, D), jnp.float32),
            pltpu.VMEM_SHARED((NP, D), jnp.float32),
            pltpu.SemaphoreType.DMA,
            pltpu.SemaphoreType.DMA,
            pltpu.SemaphoreType.DMA,
            pltpu.SemaphoreType.DMA,
            pltpu.SemaphoreType.DMA,
            pltpu.SemaphoreType.DMA,
            pltpu.SemaphoreType.DMA,
            pltpu.SemaphoreType.DMA,
            pltpu.SemaphoreType.DMA,
            pltpu.SemaphoreType.DMA,
            pltpu.SemaphoreType.DMA,
            pltpu.SemaphoreType.DMA,
        ],
    )
    def agg_kernel(
        feats_h, edges, zeros, out,
        src_v, d0, d1, d2, d3, dt, r0, r1, r2, r3, acc_sh,
        sg0, sg1, sg2, sg3, sd0, sd1, sd2, sd3, ss0, ss1, ss2, ss3,
    ):
        cid = lax.axis_index("c")
        sid = lax.axis_index("s")
        wid = cid * NS + sid
        ebase = wid * epw
        dbufs = (d0, d1, d2, d3)
        rbufs = (r0, r1, r2, r3)
        sgs = (sg0, sg1, sg2, sg3)
        sds = (sd0, sd1, sd2, sd3)
        sss = (ss0, ss1, ss2, ss3)
        pltpu.sync_copy(
            zeros.at[pl.ds(sid * rpt, rpt)], acc_sh.at[pl.ds(sid * rpt, rpt)]
        )
        pltpu.sync_copy(edges.at[pl.ds(ebase, epw)], src_v)
        plsc.subcore_barrier()

        def start_gd(i, b):
            pltpu.async_copy(
                feats_h.at[src_v.at[pl.ds(i * CH, CH)]], rbufs[b], sgs[b]
            )
            pltpu.async_copy(
                edges.at[pl.ds(E + ebase + i * CH, CH)], dbufs[b], sds[b]
            )

        def wait_gd(b):
            pltpu.make_async_copy(
                feats_h.at[src_v.at[pl.ds(0, CH)]], rbufs[b], sgs[b]
            ).wait()
            pltpu.make_async_copy(edges.at[pl.ds(E, CH)], dbufs[b], sds[b]).wait()

        def start_s(b):
            pltpu.make_async_copy(rbufs[b], acc_sh.at[dbufs[b]], sss[b]).start(
                add=True
            )

        def wait_s(b):
            pltpu.make_async_copy(rbufs[b], acc_sh.at[dbufs[b]], sss[b]).wait()

        def slot(i, b, warm):
            wait_gd(b)
            start_s(b)
            if warm:
                wait_s((b + 2) % 4)

            @pl.when(i + 2 < full)
            def _():
                start_gd(i + 2, (b + 2) % 4)

        start_gd(0, 0)
        start_gd(1, 1)
        slot(0, 0, False)
        slot(1, 1, False)
        slot(2, 2, True)
        slot(3, 3, True)

        def quad_body(q, carry):
            i = 4 * q
            slot(i, 0, True)
            slot(i + 1, 1, True)
            slot(i + 2, 2, True)
            slot(i + 3, 3, True)
            return carry

        lax.fori_loop(1, full // 4, quad_body, 0)
        slot(full - 2, 0, True)
        slot(full - 1, 1, True)
        wait_s(0)
        wait_s(1)
        if tail:
            tbase = ebase + full * CH
            pltpu.async_copy(
                feats_h.at[src_v.at[pl.ds(full * CH, tail)]],
                r0.at[pl.ds(0, tail)], sg0,
            )
            pltpu.sync_copy(edges.at[pl.ds(E + tbase, tail)], dt)
            pltpu.make_async_copy(
                feats_h.at[src_v.at[pl.ds(0, tail)]], r0.at[pl.ds(0, tail)], sg0
            ).wait()
            pltpu.sync_copy(r0.at[pl.ds(0, tail)], acc_sh.at[dt], add=True)

        plsc.subcore_barrier()
        pltpu.sync_copy(
            acc_sh.at[pl.ds(sid * rpt, rpt)],
            out.at[cid, pl.ds(sid * rpt, rpt)],
        )

    assert full % 4 == 2 and tail % 8 == 0
    zeros = jnp.zeros((NP, D), jnp.float32)
    return agg_kernel(feats, edge_flat, zeros)


_BN = 1000


def _tc_prep(x, deg_src, deg_dst):
    n_nodes, D = x.shape

    def body(x_ref, ds_ref, dd_ref, xs_ref, ns_ref, nd_ref):
        ns = lax.rsqrt(jnp.maximum(ds_ref[...], 1.0))
        nd = lax.rsqrt(jnp.maximum(dd_ref[...], 1.0))
        ns_ref[...] = ns
        nd_ref[...] = nd
        xs_ref[...] = x_ref[...] * ns

    return pl.pallas_call(
        body,
        grid=(n_nodes // _BN,),
        in_specs=[
            pl.BlockSpec((_BN, D), lambda i: (i, 0)),
            pl.BlockSpec((_BN, 1), lambda i: (i, 0)),
            pl.BlockSpec((_BN, 1), lambda i: (i, 0)),
        ],
        out_specs=[
            pl.BlockSpec((_BN, D), lambda i: (i, 0)),
            pl.BlockSpec((_BN, 1), lambda i: (i, 0)),
            pl.BlockSpec((_BN, 1), lambda i: (i, 0)),
        ],
        out_shape=[
            jax.ShapeDtypeStruct((n_nodes, D), jnp.float32),
            jax.ShapeDtypeStruct((n_nodes, 1), jnp.float32),
            jax.ShapeDtypeStruct((n_nodes, 1), jnp.float32),
        ],
    )(x, deg_src, deg_dst)


def _tc_mid(p0, p1, nd1, ns1, W1, b1, W2):
    D = p0.shape[1]
    H1 = W1.shape[1]
    H2 = W2.shape[1]
    n_nodes = nd1.shape[0]

    def body(p0_ref, p1_ref, nd_ref, ns_ref, w1_ref, b1_ref, w2_ref, o_ref):
        scaled = (p0_ref[...] + p1_ref[...]) * nd_ref[...]
        h = jnp.dot(scaled, w1_ref[...], preferred_element_type=jnp.float32)
        h = jnp.maximum(h + b1_ref[...], 0.0)
        o = jnp.dot(h, w2_ref[...], preferred_element_type=jnp.float32)
        o_ref[...] = o * ns_ref[...]

    return pl.pallas_call(
        body,
        grid=(n_nodes // _BN,),
        in_specs=[
            pl.BlockSpec((_BN, D), lambda i: (i, 0)),
            pl.BlockSpec((_BN, D), lambda i: (i, 0)),
            pl.BlockSpec((_BN, 1), lambda i: (i, 0)),
            pl.BlockSpec((_BN, 1), lambda i: (i, 0)),
            pl.BlockSpec((D, H1), lambda i: (0, 0)),
            pl.BlockSpec((1, H1), lambda i: (0, 0)),
            pl.BlockSpec((H1, H2), lambda i: (0, 0)),
        ],
        out_specs=pl.BlockSpec((_BN, H2), lambda i: (i, 0)),
        out_shape=jax.ShapeDtypeStruct((n_nodes, H2), jnp.float32),
    )(p0, p1, nd1, ns1, W1, b1.reshape(1, H1), W2)


def _tc_final(q0, q1, nd1, b2):
    H2 = q0.shape[1]
    n_nodes = nd1.shape[0]

    def body(q0_ref, q1_ref, nd_ref, b2_ref, o_ref):
        agg = (q0_ref[...] + q1_ref[...]) * nd_ref[...]
        o_ref[...] = agg + b2_ref[...]

    return pl.pallas_call(
        body,
        grid=(n_nodes // _BN,),
        in_specs=[
            pl.BlockSpec((_BN, H2), lambda i: (i, 0)),
            pl.BlockSpec((_BN, H2), lambda i: (i, 0)),
            pl.BlockSpec((_BN, 1), lambda i: (i, 0)),
            pl.BlockSpec((1, H2), lambda i: (0, 0)),
        ],
        out_specs=pl.BlockSpec((_BN, H2), lambda i: (i, 0)),
        out_shape=jax.ShapeDtypeStruct((n_nodes, H2), jnp.float32),
    )(q0, q1, nd1, b2.reshape(1, H2))


def kernel(x, edge_index, W1, b1, W2, b2):
    n = x.shape[0]
    NP = _pad_nodes(n)
    edge_flat = edge_index.reshape(-1)
    deg = _sc_degrees(edge_flat, n)
    deg_src = deg[0:n].reshape(n, 1)
    deg_dst = deg[NP:NP + n].reshape(n, 1)
    xs, ns1, nd1 = _tc_prep(x, deg_src, deg_dst)
    p = _sc_aggregate(xs, edge_flat)
    hs = _tc_mid(p[0], p[1], nd1, ns1, W1, b1, W2)
    q = _sc_aggregate(hs, edge_flat)
    return _tc_final(q[0], q[1], nd1, b2)


# speedup vs baseline: 1.0723x; 1.0723x over previous
import functools

import jax
import jax.numpy as jnp
from jax import lax
from jax.experimental import pallas as pl
from jax.experimental.pallas import tpu as pltpu
from jax.experimental.pallas import tpu_sc as plsc

NC = 2
NS = 16
CHUNK = 80


def _pad_nodes(n):
    m = NS * 16
    return (n + m - 1) // m * m


def _sc_mesh():
    return plsc.VectorSubcoreMesh(
        core_axis_name="c", subcore_axis_name="s", num_cores=NC, num_subcores=NS
    )


def _sc_degrees(edge_flat, n_nodes):
    E = edge_flat.shape[0] // 2
    ept = E // NS
    CH = 128
    full = ept // CH
    tail = ept - full * CH
    NP = _pad_nodes(n_nodes)
    rpt = NP // NS

    @functools.partial(
        pl.kernel,
        out_type=jax.ShapeDtypeStruct((NC * NP,), jnp.float32),
        mesh=_sc_mesh(),
        scratch_types=[
            pltpu.VMEM((CH,), jnp.int32),
            pltpu.VMEM((CH,), jnp.int32),
            pltpu.VMEM((CH,), jnp.int32),
            pltpu.VMEM((CH,), jnp.int32),
            pltpu.VMEM((tail,), jnp.int32) if tail else None,
            pltpu.VMEM((CH,), jnp.float32),
            pltpu.VMEM((rpt,), jnp.float32),
            pltpu.VMEM_SHARED((NP,), jnp.float32),
            pltpu.SemaphoreType.DMA,
            pltpu.SemaphoreType.DMA,
            pltpu.SemaphoreType.DMA,
            pltpu.SemaphoreType.DMA,
            pltpu.SemaphoreType.DMA,
            pltpu.SemaphoreType.DMA,
            pltpu.SemaphoreType.DMA,
            pltpu.SemaphoreType.DMA,
        ],
    )
    def deg_kernel(
        edges, out,
        i0, i1, i2, i3, it, ones_v, buf_v, deg_sh,
        sl0, sl1, sl2, sl3, ss0, ss1, ss2, ss3,
    ):
        cid = lax.axis_index("c")
        sid = lax.axis_index("s")
        ebase = cid * E + sid * ept
        ibufs = (i0, i1, i2, i3)
        slods = (sl0, sl1, sl2, sl3)
        sscat = (ss0, ss1, ss2, ss3)

        def fill(j, carry):
            ones_v[pl.ds(j * 16, 16)] = jnp.ones((16,), jnp.float32)
            return carry

        lax.fori_loop(0, CH // 16, fill, 0)

        def zbody(j, carry):
            buf_v[pl.ds(j * 16, 16)] = jnp.zeros((16,), jnp.float32)
            return carry

        lax.fori_loop(0, rpt // 16, zbody, 0)
        pltpu.sync_copy(buf_v, deg_sh.at[pl.ds(sid * rpt, rpt)])
        plsc.subcore_barrier()

        def start_l(i, b):
            pltpu.async_copy(
                edges.at[pl.ds(ebase + i * CH, CH)], ibufs[b], slods[b]
            )

        def wait_l(b):
            pltpu.make_async_copy(edges.at[pl.ds(0, CH)], ibufs[b], slods[b]).wait()

        for j in range(4):
            start_l(j, j)

        def slot(i, b):
            wait_l(b)
            pltpu.sync_copy(ones_v, deg_sh.at[ibufs[b]], add=True)

            @pl.when(i + 4 < full)
            def _():
                start_l(i + 4, b)

        def quad(q, carry):
            i = 4 * q
            slot(i, 0)
            slot(i + 1, 1)
            slot(i + 2, 2)
            slot(i + 3, 3)
            return carry

        lax.fori_loop(0, full // 4, quad, 0)
        if tail:
            pltpu.sync_copy(edges.at[pl.ds(ebase + full * CH, tail)], it)
            pltpu.sync_copy(ones_v.at[pl.ds(0, tail)], deg_sh.at[it], add=True)

        plsc.subcore_barrier()
        pltpu.sync_copy(deg_sh.at[pl.ds(sid * rpt, rpt)], buf_v)
        pltpu.sync_copy(buf_v, out.at[pl.ds(cid * NP + sid * rpt, rpt)])

    assert full % 4 == 0
    return deg_kernel(edge_flat)


def _sc_aggregate(feats, edge_flat):
    n_nodes, D = feats.shape
    E = edge_flat.shape[0] // 2
    epw = E // (NC * NS)
    CH = 128
    full = epw // CH
    tail = epw - full * CH
    NP = _pad_nodes(n_nodes)
    rpt = NP // NS

    @functools.partial(
        pl.kernel,
        out_type=jax.ShapeDtypeStruct((NC, NP, D), jnp.float32),
        mesh=_sc_mesh(),
        scratch_types=[
            pltpu.VMEM((epw,), jnp.int32),
            pltpu.VMEM((CH,), jnp.int32),
            pltpu.VMEM((CH,), jnp.int32),
            pltpu.VMEM((tail,), jnp.int32),
            pltpu.VMEM((CH, D), jnp.float32),
            pltpu.VMEM((CH, D), jnp.float32),
            pltpu.VMEM_SHARED((NP, D), jnp.float32),
            pltpu.SemaphoreType.DMA,
            pltpu.SemaphoreType.DMA,
            pltpu.SemaphoreType.DMA,
            pltpu.SemaphoreType.DMA,
            pltpu.SemaphoreType.DMA,
            pltpu.SemaphoreType.DMA,
        ],
    )
    def agg_kernel(
        feats_h, edges, zeros, out,
        src_v, d0, d1, dt, r0, r1, acc_sh, sg0, sg1, sd0, sd1, sz, sp,
    ):
        cid = lax.axis_index("c")
        sid = lax.axis_index("s")
        wid = cid * NS + sid
        ebase = wid * epw
        cz = pltpu.async_copy(
            zeros.at[pl.ds(sid * rpt, rpt)], acc_sh.at[pl.ds(sid * rpt, rpt)], sz
        )
        cp = pltpu.async_copy(edges.at[pl.ds(ebase, epw)], src_v, sp)
        cz.wait()
        cp.wait()
        plsc.subcore_barrier()

        def start_g(i, r, sem):
            pltpu.async_copy(feats_h.at[src_v.at[pl.ds(i * CH, CH)]], r, sem)

        def wait_g(r, sem):
            pltpu.make_async_copy(feats_h.at[src_v.at[pl.ds(0, CH)]], r, sem).wait()

        def start_d(i, dbuf, sem):
            pltpu.async_copy(edges.at[pl.ds(E + ebase + i * CH, CH)], dbuf, sem)

        def wait_d(dbuf, sem):
            pltpu.make_async_copy(edges.at[pl.ds(E, CH)], dbuf, sem).wait()

        start_g(0, r0, sg0)
        start_d(0, d0, sd0)
        start_g(1, r1, sg1)
        start_d(1, d1, sd1)

        def pair_body(i2, carry):
            i = 2 * i2
            wait_g(r0, sg0)
            wait_d(d0, sd0)
            pltpu.sync_copy(r0, acc_sh.at[d0], add=True)

            @pl.when(i + 2 < full)
            def _():
                start_g(i + 2, r0, sg0)
                start_d(i + 2, d0, sd0)

            wait_g(r1, sg1)
            wait_d(d1, sd1)
            pltpu.sync_copy(r1, acc_sh.at[d1], add=True)

            @pl.when(i + 3 < full)
            def _():
                start_g(i + 3, r1, sg1)
                start_d(i + 3, d1, sd1)

            return carry

        lax.fori_loop(0, full // 2, pair_body, 0)
        if tail:
            tbase = ebase + full * CH
            pltpu.async_copy(
                feats_h.at[src_v.at[pl.ds(full * CH, tail)]],
                r0.at[pl.ds(0, tail)], sg0,
            )
            pltpu.sync_copy(edges.at[pl.ds(E + tbase, tail)], dt)
            pltpu.make_async_copy(
                feats_h.at[src_v.at[pl.ds(0, tail)]], r0.at[pl.ds(0, tail)], sg0
            ).wait()
            pltpu.sync_copy(r0.at[pl.ds(0, tail)], acc_sh.at[dt], add=True)

        plsc.subcore_barrier()
        pltpu.sync_copy(
            acc_sh.at[pl.ds(sid * rpt, rpt)],
            out.at[cid, pl.ds(sid * rpt, rpt)],
        )

    assert full % 2 == 0 and tail % 8 == 0
    zeros = jnp.zeros((NP, D), jnp.float32)
    return agg_kernel(feats, edge_flat, zeros)


_BN = 1000


def _tc_prep(x, deg_src, deg_dst):
    n_nodes, D = x.shape

    def body(x_ref, ds_ref, dd_ref, xs_ref, ns_ref, nd_ref):
        ns = lax.rsqrt(jnp.maximum(ds_ref[...], 1.0))
        nd = lax.rsqrt(jnp.maximum(dd_ref[...], 1.0))
        ns_ref[...] = ns
        nd_ref[...] = nd
        xs_ref[...] = x_ref[...] * ns

    return pl.pallas_call(
        body,
        grid=(n_nodes // _BN,),
        in_specs=[
            pl.BlockSpec((_BN, D), lambda i: (i, 0)),
            pl.BlockSpec((_BN, 1), lambda i: (i, 0)),
            pl.BlockSpec((_BN, 1), lambda i: (i, 0)),
        ],
        out_specs=[
            pl.BlockSpec((_BN, D), lambda i: (i, 0)),
            pl.BlockSpec((_BN, 1), lambda i: (i, 0)),
            pl.BlockSpec((_BN, 1), lambda i: (i, 0)),
        ],
        out_shape=[
            jax.ShapeDtypeStruct((n_nodes, D), jnp.float32),
            jax.ShapeDtypeStruct((n_nodes, 1), jnp.float32),
            jax.ShapeDtypeStruct((n_nodes, 1), jnp.float32),
        ],
    )(x, deg_src, deg_dst)


def _tc_mid(p0, p1, nd1, ns1, W1, b1, W2):
    D = p0.shape[1]
    H1 = W1.shape[1]
    H2 = W2.shape[1]
    n_nodes = nd1.shape[0]

    def body(p0_ref, p1_ref, nd_ref, ns_ref, w1_ref, b1_ref, w2_ref, o_ref):
        scaled = (p0_ref[...] + p1_ref[...]) * nd_ref[...]
        h = jnp.dot(scaled, w1_ref[...], preferred_element_type=jnp.float32)
        h = jnp.maximum(h + b1_ref[...], 0.0)
        o = jnp.dot(h, w2_ref[...], preferred_element_type=jnp.float32)
        o_ref[...] = o * ns_ref[...]

    return pl.pallas_call(
        body,
        grid=(n_nodes // _BN,),
        in_specs=[
            pl.BlockSpec((_BN, D), lambda i: (i, 0)),
            pl.BlockSpec((_BN, D), lambda i: (i, 0)),
            pl.BlockSpec((_BN, 1), lambda i: (i, 0)),
            pl.BlockSpec((_BN, 1), lambda i: (i, 0)),
            pl.BlockSpec((D, H1), lambda i: (0, 0)),
            pl.BlockSpec((1, H1), lambda i: (0, 0)),
            pl.BlockSpec((H1, H2), lambda i: (0, 0)),
        ],
        out_specs=pl.BlockSpec((_BN, H2), lambda i: (i, 0)),
        out_shape=jax.ShapeDtypeStruct((n_nodes, H2), jnp.float32),
    )(p0, p1, nd1, ns1, W1, b1.reshape(1, H1), W2)


def _tc_final(q0, q1, nd1, b2):
    H2 = q0.shape[1]
    n_nodes = nd1.shape[0]

    def body(q0_ref, q1_ref, nd_ref, b2_ref, o_ref):
        agg = (q0_ref[...] + q1_ref[...]) * nd_ref[...]
        o_ref[...] = agg + b2_ref[...]

    return pl.pallas_call(
        body,
        grid=(n_nodes // _BN,),
        in_specs=[
            pl.BlockSpec((_BN, H2), lambda i: (i, 0)),
            pl.BlockSpec((_BN, H2), lambda i: (i, 0)),
            pl.BlockSpec((_BN, 1), lambda i: (i, 0)),
            pl.BlockSpec((1, H2), lambda i: (0, 0)),
        ],
        out_specs=pl.BlockSpec((_BN, H2), lambda i: (i, 0)),
        out_shape=jax.ShapeDtypeStruct((n_nodes, H2), jnp.float32),
    )(q0, q1, nd1, b2.reshape(1, H2))


def kernel(x, edge_index, W1, b1, W2, b2):
    n = x.shape[0]
    NP = _pad_nodes(n)
    edge_flat = edge_index.reshape(-1)
    deg = _sc_degrees(edge_flat, n)
    deg_src = deg[0:n].reshape(n, 1)
    deg_dst = deg[NP:NP + n].reshape(n, 1)
    xs, ns1, nd1 = _tc_prep(x, deg_src, deg_dst)
    p = _sc_aggregate(xs, edge_flat)
    hs = _tc_mid(p[0], p[1], nd1, ns1, W1, b1, W2)
    q = _sc_aggregate(hs, edge_flat)
    return _tc_final(q[0], q[1], nd1, b2)
